# Initial kernel scaffold; baseline (speedup 1.0000x reference)
import jax
import jax.numpy as jnp
from jax.experimental import pallas as pl


def kernel(x, edge_index, W1_l, W1_r, b1, g1, be1, W2_l, W2_r, b2, g2, be2, Wl1, bl1, Wl2, bl2, Wh, bh):
    raise NotImplementedError("write your pallas kernel here")


# speedup vs baseline: 5.9165x; 5.9165x over previous
import functools

import jax
import jax.numpy as jnp
from jax import lax
from jax.experimental import pallas as pl
from jax.experimental.pallas import tpu as pltpu
from jax.experimental.pallas import tpu_sc as plsc

N = 10000
D = 128
E = 320000

NSC = 2
NTILE = 16
HD = D // NSC
CHUNK = 128
CHUNKS = (E + NTILE * CHUNK - 1) // (NTILE * CHUNK)
EPAD = NTILE * CHUNK * CHUNKS
NPAD = 10112
RPT = NPAD // NTILE
CW = 8


def _sc_agg_body(with_cnt, *refs):
    if with_cnt:
        (src_g, dst_g, tab, zrow, zcnt, ones_h,
         agg_out, cnt_out,
         src_v, dst_v, rows_v, ones_v, agg_sh, cnt_sh, sem) = refs
    else:
        (src_g, dst_g, tab, zrow,
         agg_out,
         src_v, dst_v, rows_v, agg_sh, sem) = refs

    cid = lax.axis_index("c")
    sid = lax.axis_index("s")
    r0 = sid * RPT

    pltpu.sync_copy(zrow.at[pl.ds(r0, RPT)], agg_sh.at[pl.ds(r0, RPT)])
    if with_cnt:
        pltpu.sync_copy(zcnt.at[pl.ds(r0, RPT)], cnt_sh.at[pl.ds(r0, RPT)])
        pltpu.sync_copy(ones_h, ones_v)
    pltpu.sync_copy(src_g.at[cid, sid], src_v)
    pltpu.sync_copy(dst_g.at[sid], dst_v)
    plsc.subcore_barrier()

    def chunk_step(j, carry):
        pltpu.async_copy(tab.at[src_v.at[j]], rows_v, sem).wait()
        pltpu.sync_copy(rows_v, agg_sh.at[dst_v.at[j]], add=True)
        if with_cnt:
            pltpu.sync_copy(ones_v, cnt_sh.at[dst_v.at[j]], add=True)
        return carry

    lax.fori_loop(0, CHUNKS, chunk_step, 0)
    plsc.subcore_barrier()

    pltpu.sync_copy(agg_sh.at[pl.ds(r0, RPT)], agg_out.at[cid, pl.ds(r0, RPT)])
    if with_cnt:
        pltpu.sync_copy(cnt_sh.at[pl.ds(r0, RPT)],
                        cnt_out.at[cid, pl.ds(r0, RPT)])


def _make_sc_agg(with_cnt):
    mesh = plsc.VectorSubcoreMesh(core_axis_name="c", subcore_axis_name="s")
    out_type = [jax.ShapeDtypeStruct((NSC, NPAD, HD), jnp.float32)]
    scratch = [
        pltpu.VMEM((CHUNKS, CHUNK), jnp.int32),
        pltpu.VMEM((CHUNKS, CHUNK), jnp.int32),
        pltpu.VMEM((CHUNK, HD), jnp.float32),
    ]
    if with_cnt:
        out_type.append(jax.ShapeDtypeStruct((NSC, NPAD, CW), jnp.float32))
        scratch.append(pltpu.VMEM((CHUNK, CW), jnp.float32))
    scratch.append(pltpu.VMEM_SHARED((NPAD, HD), jnp.float32))
    if with_cnt:
        scratch.append(pltpu.VMEM_SHARED((NPAD, CW), jnp.float32))
    scratch.append(pltpu.SemaphoreType.DMA)
    return pl.kernel(
        functools.partial(_sc_agg_body, with_cnt),
        out_type=tuple(out_type) if len(out_type) > 1 else out_type[0],
        mesh=mesh,
        scratch_types=scratch,
        compiler_params=pltpu.CompilerParams(use_tc_tiling_on_sc=False),
    )


_sc_agg_cnt = _make_sc_agg(True)
_sc_agg = _make_sc_agg(False)


def _bn_relu(y, g, b):
    mu = jnp.mean(y, axis=0, keepdims=True)
    var = jnp.mean((y - mu) * (y - mu), axis=0, keepdims=True)
    return jnp.maximum(g * (y - mu) * lax.rsqrt(var + 1e-5) + b, 0.0)


def _tc1_body(aggp, cntp, x, w1l_t, w1r_t, b1, g1, be1, h_out):
    agg = jnp.concatenate([aggp[0, :N, :], aggp[1, :N, :]], axis=-1)
    cnt = cntp[0, :N, 0:1]
    mean = agg * (1.0 / jnp.maximum(cnt, 1.0))
    y = (jnp.dot(mean, w1l_t[...], preferred_element_type=jnp.float32)
         + jnp.dot(x[...], w1r_t[...], preferred_element_type=jnp.float32)
         + b1[...])
    h = _bn_relu(y, g1[...], be1[...])
    h_out[0] = h[:, :HD]
    h_out[1] = h[:, HD:]


def _tc2_body(aggp, cntp, hs, w2l_t, w2r_t, b2, g2, be2,
              wl1_t, bl1, wl2_t, bl2, wh_t, bh, out):
    agg = jnp.concatenate([aggp[0, :N, :], aggp[1, :N, :]], axis=-1)
    h1 = jnp.concatenate([hs[0], hs[1]], axis=-1)
    cnt = cntp[0, :N, 0:1]
    mean = agg * (1.0 / jnp.maximum(cnt, 1.0))
    y = (jnp.dot(mean, w2l_t[...], preferred_element_type=jnp.float32)
         + jnp.dot(h1, w2r_t[...], preferred_element_type=jnp.float32)
         + b2[...])
    h = _bn_relu(y, g2[...], be2[...])
    h = jnp.maximum(jnp.dot(h, wl1_t[...], preferred_element_type=jnp.float32)
                    + bl1[...], 0.0)
    h = jnp.maximum(jnp.dot(h, wl2_t[...], preferred_element_type=jnp.float32)
                    + bl2[...], 0.0)
    out[...] = (jnp.dot(h, wh_t[...], preferred_element_type=jnp.float32)
                + bh[...])


_tc1 = pl.pallas_call(
    _tc1_body,
    out_shape=jax.ShapeDtypeStruct((NSC, N, HD), jnp.float32),
)

_tc2 = pl.pallas_call(
    _tc2_body,
    out_shape=jax.ShapeDtypeStruct((N, 8), jnp.float32),
)


def kernel(x, edge_index, W1_l, W1_r, b1, g1, be1, W2_l, W2_r, b2, g2, be2,
           Wl1, bl1, Wl2, bl2, Wh, bh):
    src = edge_index[0]
    dst = edge_index[1]
    pad = EPAD - E
    src_p = jnp.concatenate([src, jnp.zeros((pad,), jnp.int32)])
    dst_p = jnp.concatenate([dst, jnp.full((pad,), N, jnp.int32)])
    src_g = jnp.stack([src_p, src_p + N]).reshape(NSC, NTILE, CHUNKS, CHUNK)
    dst_g = dst_p.reshape(NTILE, CHUNKS, CHUNK)

    xs = x.reshape(N, NSC, HD).transpose(1, 0, 2).reshape(NSC * N, HD)

    zrow = jnp.zeros((NPAD, HD), jnp.float32)
    zcnt = jnp.zeros((NPAD, CW), jnp.float32)
    ones_h = jnp.ones((CHUNK, CW), jnp.float32)

    aggp1, cntp = _sc_agg_cnt(src_g, dst_g, xs, zrow, zcnt, ones_h)
    hs = _tc1(aggp1, cntp, x, W1_l.T, W1_r.T, b1, g1, be1)
    aggp2 = _sc_agg(src_g, dst_g, hs.reshape(NSC * N, HD), zrow)

    wh_t = jnp.zeros((D, 8), jnp.float32).at[:, 0].set(Wh[0])
    bh8 = jnp.zeros((8,), jnp.float32).at[0].set(bh[0])
    out8 = _tc2(aggp2, cntp, hs, W2_l.T, W2_r.T, b2, g2, be2,
                Wl1.T, bl1, Wl2.T, bl2, wh_t, bh8)
    return out8[:, :1]


# speedup vs baseline: 6.0648x; 1.0251x over previous
import functools

import jax
import jax.numpy as jnp
from jax import lax
from jax.experimental import pallas as pl
from jax.experimental.pallas import tpu as pltpu
from jax.experimental.pallas import tpu_sc as plsc

N = 10000
D = 128
E = 320000

NSC = 2
NTILE = 16
HD = D // NSC
CHUNK = 128
CHUNKS = (E + NTILE * CHUNK - 1) // (NTILE * CHUNK)
EPAD = NTILE * CHUNK * CHUNKS
NPAD = 10112
RPT = NPAD // NTILE
CW = 8


def _sc_agg_body(with_cnt, *refs):
    if with_cnt:
        (src_g, dst_g, tab, zrow, zcnt, ones_h,
         agg_out, cnt_out,
         src_v, dst_v, rows_v, ones_v, agg_sh, cnt_sh, sem) = refs
    else:
        (src_g, dst_g, tab, zrow,
         agg_out,
         src_v, dst_v, rows_v, agg_sh, sem) = refs

    cid = lax.axis_index("c")
    sid = lax.axis_index("s")
    r0 = sid * RPT

    pltpu.sync_copy(zrow.at[pl.ds(r0, RPT)], agg_sh.at[pl.ds(r0, RPT)])
    if with_cnt:
        pltpu.sync_copy(zcnt.at[pl.ds(r0, RPT)], cnt_sh.at[pl.ds(r0, RPT)])
        pltpu.sync_copy(ones_h, ones_v)
    pltpu.sync_copy(src_g.at[cid, sid], src_v)
    pltpu.sync_copy(dst_g.at[sid], dst_v)
    plsc.subcore_barrier()

    def chunk_step(j, carry):
        pltpu.async_copy(tab.at[src_v.at[j]], rows_v, sem).wait()
        pltpu.sync_copy(rows_v, agg_sh.at[dst_v.at[j]], add=True)
        if with_cnt:
            pltpu.sync_copy(ones_v, cnt_sh.at[dst_v.at[j]], add=True)
        return carry

    lax.fori_loop(0, CHUNKS, chunk_step, 0)
    plsc.subcore_barrier()

    pltpu.sync_copy(agg_sh.at[pl.ds(r0, RPT)], agg_out.at[cid, pl.ds(r0, RPT)])
    if with_cnt:
        pltpu.sync_copy(cnt_sh.at[pl.ds(r0, RPT)],
                        cnt_out.at[cid, pl.ds(r0, RPT)])


def _make_sc_agg(with_cnt):
    mesh = plsc.VectorSubcoreMesh(core_axis_name="c", subcore_axis_name="s")
    out_type = [jax.ShapeDtypeStruct((NSC, NPAD, HD), jnp.float32)]
    scratch = [
        pltpu.VMEM((CHUNKS, CHUNK), jnp.int32),
        pltpu.VMEM((CHUNKS, CHUNK), jnp.int32),
        pltpu.VMEM((CHUNK, HD), jnp.float32),
    ]
    if with_cnt:
        out_type.append(jax.ShapeDtypeStruct((NSC, NPAD, CW), jnp.float32))
        scratch.append(pltpu.VMEM((CHUNK, CW), jnp.float32))
    scratch.append(pltpu.VMEM_SHARED((NPAD, HD), jnp.float32))
    if with_cnt:
        scratch.append(pltpu.VMEM_SHARED((NPAD, CW), jnp.float32))
    scratch.append(pltpu.SemaphoreType.DMA)
    return pl.kernel(
        functools.partial(_sc_agg_body, with_cnt),
        out_type=tuple(out_type) if len(out_type) > 1 else out_type[0],
        mesh=mesh,
        scratch_types=scratch,
        compiler_params=pltpu.CompilerParams(use_tc_tiling_on_sc=False),
    )


_sc_agg_cnt = _make_sc_agg(True)
_sc_agg = _make_sc_agg(False)


def _bn_relu(y, g, b):
    mu = jnp.mean(y, axis=0, keepdims=True)
    var = jnp.mean((y - mu) * (y - mu), axis=0, keepdims=True)
    return jnp.maximum(g * (y - mu) * lax.rsqrt(var + 1e-5) + b, 0.0)


def _tc1_body(aggp, cntp, x, w1l_t, w1r_t, b1, g1, be1, h_out):
    agg = jnp.concatenate([aggp[0, :N, :], aggp[1, :N, :]], axis=-1)
    cnt = cntp[0, :N, 0:1]
    mean = agg * (1.0 / jnp.maximum(cnt, 1.0))
    y = (jnp.dot(mean, w1l_t[...], preferred_element_type=jnp.float32)
         + jnp.dot(x[...], w1r_t[...], preferred_element_type=jnp.float32)
         + b1[...])
    h_out[...] = _bn_relu(y, g1[...], be1[...])


def _tc2_body(aggp, cntp, h1, w2l_t, w2r_t, b2, g2, be2,
              wl1_t, bl1, wl2_t, bl2, wh_t, bh, out):
    agg = jnp.concatenate([aggp[0, :N, :], aggp[1, :N, :]], axis=-1)
    cnt = cntp[0, :N, 0:1]
    mean = agg * (1.0 / jnp.maximum(cnt, 1.0))
    y = (jnp.dot(mean, w2l_t[...], preferred_element_type=jnp.float32)
         + jnp.dot(h1[...], w2r_t[...], preferred_element_type=jnp.float32)
         + b2[...])
    h = _bn_relu(y, g2[...], be2[...])
    h = jnp.maximum(jnp.dot(h, wl1_t[...], preferred_element_type=jnp.float32)
                    + bl1[...], 0.0)
    h = jnp.maximum(jnp.dot(h, wl2_t[...], preferred_element_type=jnp.float32)
                    + bl2[...], 0.0)
    out[...] = (jnp.dot(h, wh_t[...], preferred_element_type=jnp.float32)
                + bh[...])


_tc1 = pl.pallas_call(
    _tc1_body,
    out_shape=jax.ShapeDtypeStruct((N, D), jnp.float32),
)

_tc2 = pl.pallas_call(
    _tc2_body,
    out_shape=jax.ShapeDtypeStruct((N, 8), jnp.float32),
)


def kernel(x, edge_index, W1_l, W1_r, b1, g1, be1, W2_l, W2_r, b2, g2, be2,
           Wl1, bl1, Wl2, bl2, Wh, bh):
    src = edge_index[0]
    dst = edge_index[1]
    pad = EPAD - E
    src_p = jnp.concatenate([src, jnp.zeros((pad,), jnp.int32)])
    dst_p = jnp.concatenate([dst, jnp.full((pad,), N, jnp.int32)])
    src2 = src_p * 2
    src_g = jnp.stack([src2, src2 + 1]).reshape(NSC, NTILE, CHUNKS, CHUNK)
    dst_g = dst_p.reshape(NTILE, CHUNKS, CHUNK)

    zrow = jnp.zeros((NPAD, HD), jnp.float32)
    zcnt = jnp.zeros((NPAD, CW), jnp.float32)
    ones_h = jnp.ones((CHUNK, CW), jnp.float32)

    aggp1, cntp = _sc_agg_cnt(src_g, dst_g, x.reshape(NSC * N, HD),
                              zrow, zcnt, ones_h)
    h1 = _tc1(aggp1, cntp, x, W1_l.T, W1_r.T, b1, g1, be1)
    aggp2 = _sc_agg(src_g, dst_g, h1.reshape(NSC * N, HD), zrow)

    wh_t = jnp.zeros((D, 8), jnp.float32).at[:, 0].set(Wh[0])
    bh8 = jnp.zeros((8,), jnp.float32).at[0].set(bh[0])
    out8 = _tc2(aggp2, cntp, h1, W2_l.T, W2_r.T, b2, g2, be2,
                Wl1.T, bl1, Wl2.T, bl2, wh_t, bh8)
    return out8[:, :1]
